# baseline (speedup 1.0000x reference)
import functools

import jax
import jax.numpy as jnp
from jax import lax
from jax.experimental import pallas as pl
from jax.experimental.pallas import tpu as pltpu
from jax.experimental.pallas import tpu_sc as plsc

B = 16384
F = 32


def _transpose_body(x_ref, o_ref):
  eye = jnp.eye(F, dtype=jnp.float32)
  o_ref[...] = jax.lax.dot_general(
      x_ref[...], eye, (((0,), (0,)), ((), ())),
      preferred_element_type=jnp.float32)


def _to_row_major(table):
  n = table.shape[0]
  tt = table.T
  blk = 8192
  grid = (pl.cdiv(n, blk),)
  return pl.pallas_call(
      _transpose_body,
      grid=grid,
      in_specs=[pl.BlockSpec((F, blk), lambda i: (0, i))],
      out_specs=pl.BlockSpec((blk, F), lambda i: (i, 0)),
      out_shape=jax.ShapeDtypeStruct((n, F), jnp.float32),
  )(tt)


@functools.lru_cache(maxsize=None)
def _make_sc_gather(nc: int, ns: int, b_per_w: int):
  mesh = plsc.VectorSubcoreMesh(core_axis_name="c", subcore_axis_name="s")

  @functools.partial(
      pl.kernel,
      mesh=mesh,
      out_type=(
          jax.ShapeDtypeStruct((B, F), jnp.float32),
          jax.ShapeDtypeStruct((B, F), jnp.float32),
          jax.ShapeDtypeStruct((B, F), jnp.float32),
      ),
      scratch_types=[
          pltpu.VMEM((b_per_w,), jnp.int32),
          pltpu.VMEM((b_per_w,), jnp.int32),
          pltpu.VMEM((b_per_w, F), jnp.float32),
          pltpu.VMEM((b_per_w, F), jnp.float32),
          pltpu.VMEM((b_per_w, F), jnp.float32),
          pltpu.VMEM((b_per_w, F), jnp.float32),
          pltpu.SemaphoreType.DMA,
      ],
      compiler_params=pltpu.CompilerParams(use_tc_tiling_on_sc=False),
  )
  def sc_gather(uidx_hbm, iidx_hbm, gu_hbm, gi_hbm, mu_hbm, mi_hbm,
                gmf_out, mlpu_out, mlpi_out,
                uidx_v, iidx_v, gu_v, gi_v, mu_v, mi_v, sem):
    wid = lax.axis_index("s") * nc + lax.axis_index("c")
    base = wid * b_per_w
    pltpu.sync_copy(uidx_hbm.at[pl.ds(base, b_per_w)], uidx_v)
    pltpu.sync_copy(iidx_hbm.at[pl.ds(base, b_per_w)], iidx_v)
    c1 = pltpu.async_copy(gu_hbm.at[uidx_v], gu_v, sem)
    c2 = pltpu.async_copy(gi_hbm.at[iidx_v], gi_v, sem)
    c3 = pltpu.async_copy(mu_hbm.at[uidx_v], mu_v, sem)
    c4 = pltpu.async_copy(mi_hbm.at[iidx_v], mi_v, sem)
    c1.wait()
    c2.wait()
    c3.wait()
    c4.wait()

    def row(i, carry):
      for j in range(F // 16):
        s = pl.ds(j * 16, 16)
        gu_v[i, s] = gu_v[i, s] * gi_v[i, s]
      return carry

    lax.fori_loop(0, b_per_w, row, 0)

    pltpu.sync_copy(gu_v, gmf_out.at[pl.ds(base, b_per_w)])
    pltpu.sync_copy(mu_v, mlpu_out.at[pl.ds(base, b_per_w)])
    pltpu.sync_copy(mi_v, mlpi_out.at[pl.ds(base, b_per_w)])

  return sc_gather


def _tc_mlp_body(mu_ref, mi_ref, gmf_ref, w1a_ref, w1b_ref, b1_ref,
                 w2_ref, b2_ref, wog_ref, wom_ref, bo_ref, out_ref):
  h = jnp.dot(mu_ref[...], w1a_ref[...], preferred_element_type=jnp.float32)
  h = h + jnp.dot(mi_ref[...], w1b_ref[...], preferred_element_type=jnp.float32)
  h = jnp.maximum(h + b1_ref[...], 0.0)
  h2 = jnp.dot(h, w2_ref[...], preferred_element_type=jnp.float32)
  h2 = jnp.maximum(h2 + b2_ref[...], 0.0)
  out = jnp.sum(gmf_ref[...] * wog_ref[...], axis=1)
  out = out + jnp.sum(h2 * wom_ref[...], axis=1)
  out_ref[...] = out + bo_ref[0]


def _tc_mlp(mlp_u, mlp_i, gmf, W1a, W1b, b1, W2, b2, wo_g, wo_m, bo):
  blk = 2048
  grid = (B // blk,)
  row_spec = pl.BlockSpec((blk, F), lambda i: (i, 0))
  full = lambda shape: pl.BlockSpec(shape, lambda i: tuple(0 for _ in shape))
  return pl.pallas_call(
      _tc_mlp_body,
      grid=grid,
      in_specs=[
          row_spec, row_spec, row_spec,
          full(W1a.shape), full(W1b.shape), full(b1.shape),
          full(W2.shape), full(b2.shape),
          full(wo_g.shape), full(wo_m.shape), full(bo.shape),
      ],
      out_specs=pl.BlockSpec((blk,), lambda i: (i,)),
      out_shape=jax.ShapeDtypeStruct((B,), jnp.float32),
  )(mlp_u, mlp_i, gmf, W1a, W1b, b1, W2, b2, wo_g, wo_m, bo)


@jax.jit
def _neumf(user_idx, item_idx, gmf_user_emb, gmf_item_emb,
           mlp_user_emb, mlp_item_emb, W1, b1, W2, b2, Wo, bo):
  info = plsc.get_sparse_core_info()
  nw = info.num_cores * info.num_subcores
  sc = _make_sc_gather(info.num_cores, info.num_subcores, B // nw)
  gu = _to_row_major(gmf_user_emb)
  gi = _to_row_major(gmf_item_emb)
  mu = _to_row_major(mlp_user_emb)
  mi = _to_row_major(mlp_item_emb)
  gmf, mlp_u, mlp_i = sc(user_idx.astype(jnp.int32),
                         item_idx.astype(jnp.int32), gu, gi, mu, mi)
  W1a, W1b = W1[:F], W1[F:]
  wo_g, wo_m = Wo[:F, 0], Wo[F:, 0]
  return _tc_mlp(mlp_u, mlp_i, gmf, W1a, W1b, b1, W2, b2, wo_g, wo_m, bo)


def kernel(user_idx, item_idx, gmf_user_emb, gmf_item_emb,
           mlp_user_emb, mlp_item_emb, W1, b1, W2, b2, Wo, bo):
  return _neumf(user_idx, item_idx, gmf_user_emb, gmf_item_emb,
                mlp_user_emb, mlp_item_emb, W1, b1, W2, b2, Wo, bo)


# speedup vs baseline: 1.0724x; 1.0724x over previous
import functools

import jax
import jax.numpy as jnp
from jax import lax
from jax.experimental import pallas as pl
from jax.experimental.pallas import tpu as pltpu
from jax.experimental.pallas import tpu_sc as plsc

B = 16384
F = 32


def _transpose_body(x_ref, o_ref):
  eye = jnp.eye(F, dtype=jnp.float32)
  o_ref[...] = jax.lax.dot_general(
      x_ref[...], eye, (((0,), (0,)), ((), ())),
      preferred_element_type=jnp.float32)


def _to_row_major(table):
  n = table.shape[0]
  tt = table.T
  blk = 32768
  grid = (pl.cdiv(n, blk),)
  return pl.pallas_call(
      _transpose_body,
      grid=grid,
      in_specs=[pl.BlockSpec((F, blk), lambda i: (0, i))],
      out_specs=pl.BlockSpec((blk, F), lambda i: (i, 0)),
      out_shape=jax.ShapeDtypeStruct((n, F), jnp.float32),
  )(tt)


@functools.lru_cache(maxsize=None)
def _make_sc_gather(nc: int, ns: int, b_per_w: int):
  mesh = plsc.VectorSubcoreMesh(core_axis_name="c", subcore_axis_name="s")

  @functools.partial(
      pl.kernel,
      mesh=mesh,
      out_type=(
          jax.ShapeDtypeStruct((B, F), jnp.float32),
          jax.ShapeDtypeStruct((B, F), jnp.float32),
          jax.ShapeDtypeStruct((B, F), jnp.float32),
      ),
      scratch_types=[
          pltpu.VMEM((b_per_w,), jnp.int32),
          pltpu.VMEM((b_per_w,), jnp.int32),
          pltpu.VMEM((b_per_w, F), jnp.float32),
          pltpu.VMEM((b_per_w, F), jnp.float32),
          pltpu.VMEM((b_per_w, F), jnp.float32),
          pltpu.VMEM((b_per_w, F), jnp.float32),
          pltpu.SemaphoreType.DMA,
      ],
      compiler_params=pltpu.CompilerParams(use_tc_tiling_on_sc=False),
  )
  def sc_gather(uidx_hbm, iidx_hbm, gu_hbm, gi_hbm, mu_hbm, mi_hbm,
                gmf_out, mlpu_out, mlpi_out,
                uidx_v, iidx_v, gu_v, gi_v, mu_v, mi_v, sem):
    wid = lax.axis_index("s") * nc + lax.axis_index("c")
    base = wid * b_per_w
    pltpu.sync_copy(uidx_hbm.at[pl.ds(base, b_per_w)], uidx_v)
    pltpu.sync_copy(iidx_hbm.at[pl.ds(base, b_per_w)], iidx_v)
    c1 = pltpu.async_copy(gu_hbm.at[uidx_v], gu_v, sem)
    c2 = pltpu.async_copy(gi_hbm.at[iidx_v], gi_v, sem)
    c3 = pltpu.async_copy(mu_hbm.at[uidx_v], mu_v, sem)
    c4 = pltpu.async_copy(mi_hbm.at[iidx_v], mi_v, sem)
    c1.wait()
    c2.wait()
    c3.wait()
    c4.wait()

    def row(i, carry):
      for j in range(F // 16):
        s = pl.ds(j * 16, 16)
        gu_v[i, s] = gu_v[i, s] * gi_v[i, s]
      return carry

    lax.fori_loop(0, b_per_w, row, 0)

    pltpu.sync_copy(gu_v, gmf_out.at[pl.ds(base, b_per_w)])
    pltpu.sync_copy(mu_v, mlpu_out.at[pl.ds(base, b_per_w)])
    pltpu.sync_copy(mi_v, mlpi_out.at[pl.ds(base, b_per_w)])

  return sc_gather


def _tc_mlp_body(mu_ref, mi_ref, gmf_ref, w1a_ref, w1b_ref, b1_ref,
                 w2_ref, b2_ref, wog_ref, wom_ref, bo_ref, out_ref):
  h = jnp.dot(mu_ref[...], w1a_ref[...], preferred_element_type=jnp.float32)
  h = h + jnp.dot(mi_ref[...], w1b_ref[...], preferred_element_type=jnp.float32)
  h = jnp.maximum(h + b1_ref[...], 0.0)
  h2 = jnp.dot(h, w2_ref[...], preferred_element_type=jnp.float32)
  h2 = jnp.maximum(h2 + b2_ref[...], 0.0)
  out = jnp.sum(gmf_ref[...] * wog_ref[...], axis=1)
  out = out + jnp.sum(h2 * wom_ref[...], axis=1)
  out_ref[...] = out + bo_ref[0]


def _tc_mlp(mlp_u, mlp_i, gmf, W1a, W1b, b1, W2, b2, wo_g, wo_m, bo):
  blk = 2048
  grid = (B // blk,)
  row_spec = pl.BlockSpec((blk, F), lambda i: (i, 0))
  full = lambda shape: pl.BlockSpec(shape, lambda i: tuple(0 for _ in shape))
  return pl.pallas_call(
      _tc_mlp_body,
      grid=grid,
      in_specs=[
          row_spec, row_spec, row_spec,
          full(W1a.shape), full(W1b.shape), full(b1.shape),
          full(W2.shape), full(b2.shape),
          full(wo_g.shape), full(wo_m.shape), full(bo.shape),
      ],
      out_specs=pl.BlockSpec((blk,), lambda i: (i,)),
      out_shape=jax.ShapeDtypeStruct((B,), jnp.float32),
  )(mlp_u, mlp_i, gmf, W1a, W1b, b1, W2, b2, wo_g, wo_m, bo)


@jax.jit
def _neumf(user_idx, item_idx, gmf_user_emb, gmf_item_emb,
           mlp_user_emb, mlp_item_emb, W1, b1, W2, b2, Wo, bo):
  info = plsc.get_sparse_core_info()
  nw = info.num_cores * info.num_subcores
  sc = _make_sc_gather(info.num_cores, info.num_subcores, B // nw)
  gu = _to_row_major(gmf_user_emb)
  gi = _to_row_major(gmf_item_emb)
  mu = _to_row_major(mlp_user_emb)
  mi = _to_row_major(mlp_item_emb)
  gmf, mlp_u, mlp_i = sc(user_idx.astype(jnp.int32),
                         item_idx.astype(jnp.int32), gu, gi, mu, mi)
  W1a, W1b = W1[:F], W1[F:]
  wo_g, wo_m = Wo[:F, 0], Wo[F:, 0]
  return _tc_mlp(mlp_u, mlp_i, gmf, W1a, W1b, b1, W2, b2, wo_g, wo_m, bo)


def kernel(user_idx, item_idx, gmf_user_emb, gmf_item_emb,
           mlp_user_emb, mlp_item_emb, W1, b1, W2, b2, Wo, bo):
  return _neumf(user_idx, item_idx, gmf_user_emb, gmf_item_emb,
                mlp_user_emb, mlp_item_emb, W1, b1, W2, b2, Wo, bo)


# speedup vs baseline: 2.0051x; 1.8697x over previous
import functools

import jax
import jax.numpy as jnp
from jax import lax
from jax.experimental import pallas as pl
from jax.experimental.pallas import tpu as pltpu
from jax.experimental.pallas import tpu_sc as plsc

B = 16384
F = 32
NQ = 4


RBLK = 8192


def _relayout_body(x_ref, o_ref):
  eye = jnp.eye(F, dtype=jnp.float32)
  x = x_ref[...]
  parts = []
  for a in range(NQ):
    parts.append(jax.lax.dot_general(
        x[:, a * RBLK:(a + 1) * RBLK], eye, (((0,), (0,)), ((), ())),
        preferred_element_type=jnp.float32))
  o_ref[...] = jnp.concatenate(parts, axis=1)


def _relayout(table):
  n = table.shape[0]
  tt = table.T
  nblk = pl.cdiv(n, NQ * RBLK)
  return pl.pallas_call(
      _relayout_body,
      grid=(nblk,),
      in_specs=[pl.BlockSpec((F, NQ * RBLK), lambda i: (0, i))],
      out_specs=pl.BlockSpec((RBLK, NQ * F), lambda i: (i, 0)),
      out_shape=jax.ShapeDtypeStruct((nblk * RBLK, NQ * F), jnp.float32),
  )(tt)


CHUNK = 256


@functools.lru_cache(maxsize=None)
def _make_sc_gather(nc: int, ns: int, b_per_w: int):
  mesh = plsc.VectorSubcoreMesh(core_axis_name="c", subcore_axis_name="s")

  @functools.partial(
      pl.kernel,
      mesh=mesh,
      out_type=tuple(
          jax.ShapeDtypeStruct((B, NQ * F), jnp.float32) for _ in range(4)),
      scratch_types=[
          pltpu.VMEM((b_per_w,), jnp.int32),
          pltpu.VMEM((b_per_w,), jnp.int32),
          pltpu.VMEM((CHUNK, NQ * F), jnp.float32),
          pltpu.VMEM((CHUNK, NQ * F), jnp.float32),
          pltpu.SemaphoreType.DMA,
          pltpu.SemaphoreType.DMA,
      ],
      compiler_params=pltpu.CompilerParams(use_tc_tiling_on_sc=False),
  )
  def sc_gather(urow_hbm, irow_hbm, gu_hbm, gi_hbm, mu_hbm, mi_hbm,
                gu_out, gi_out, mu_out, mi_out,
                urow_v, irow_v, buf0_v, buf1_v, sem0, sem1):
    wid = lax.axis_index("s") * nc + lax.axis_index("c")
    base = wid * b_per_w
    pltpu.sync_copy(urow_hbm.at[pl.ds(base, b_per_w)], urow_v)
    pltpu.sync_copy(irow_hbm.at[pl.ds(base, b_per_w)], irow_v)

    n_chunks = b_per_w // CHUNK
    work = []
    for table, row_v, out in ((gu_hbm, urow_v, gu_out),
                              (gi_hbm, irow_v, gi_out),
                              (mu_hbm, urow_v, mu_out),
                              (mi_hbm, irow_v, mi_out)):
      for c in range(n_chunks):
        work.append((table, row_v, out, c))

    bufs = (buf0_v, buf1_v)
    sems = (sem0, sem1)
    copies = [None, None]
    for k, (table, row_v, out, c) in enumerate(work):
      slot = k % 2
      if copies[slot] is not None:
        pt, pr, pout, pc, pcopy = copies[slot]
        pcopy.wait()
        pltpu.sync_copy(bufs[slot], pout.at[pl.ds(base + pc * CHUNK, CHUNK)])
      idx_slice = row_v.at[pl.ds(c * CHUNK, CHUNK)]
      cp = pltpu.async_copy(table.at[idx_slice], bufs[slot], sems[slot])
      copies[slot] = (table, row_v, out, c, cp)
    for slot in range(2):
      if copies[slot] is not None:
        pt, pr, pout, pc, pcopy = copies[slot]
        pcopy.wait()
        pltpu.sync_copy(bufs[slot], pout.at[pl.ds(base + pc * CHUNK, CHUNK)])

  return sc_gather


def _pick(w128, quarter):
  out = None
  for a in range(NQ):
    part = jnp.where(quarter == a, w128[:, a * F:(a + 1) * F], 0.0)
    out = part if out is None else out + part
  return out


def _tc_mlp_body(uq_ref, iq_ref, gu_ref, gi_ref, mu_ref, mi_ref,
                 w1a_ref, w1b_ref, b1_ref, w2_ref, b2_ref,
                 wog_ref, wom_ref, bo_ref, out_ref):
  uq = uq_ref[...]
  iq = iq_ref[...]
  gu = _pick(gu_ref[...], uq)
  gi = _pick(gi_ref[...], iq)
  mu = _pick(mu_ref[...], uq)
  mi = _pick(mi_ref[...], iq)
  gmf = gu * gi
  h = jnp.dot(mu, w1a_ref[...], preferred_element_type=jnp.float32)
  h = h + jnp.dot(mi, w1b_ref[...], preferred_element_type=jnp.float32)
  h = jnp.maximum(h + b1_ref[...], 0.0)
  h2 = jnp.dot(h, w2_ref[...], preferred_element_type=jnp.float32)
  h2 = jnp.maximum(h2 + b2_ref[...], 0.0)
  out = jnp.sum(gmf * wog_ref[...], axis=1)
  out = out + jnp.sum(h2 * wom_ref[...], axis=1)
  out_ref[...] = out + bo_ref[0]


def _tc_mlp(uq, iq, gu, gi, mu, mi, W1a, W1b, b1, W2, b2, wo_g, wo_m, bo):
  blk = 2048
  grid = (B // blk,)
  idx_spec = pl.BlockSpec((blk, 1), lambda i: (i, 0))
  row_spec = pl.BlockSpec((blk, NQ * F), lambda i: (i, 0))
  full = lambda shape: pl.BlockSpec(shape, lambda i: tuple(0 for _ in shape))
  return pl.pallas_call(
      _tc_mlp_body,
      grid=grid,
      in_specs=[
          idx_spec, idx_spec,
          row_spec, row_spec, row_spec, row_spec,
          full(W1a.shape), full(W1b.shape), full(b1.shape),
          full(W2.shape), full(b2.shape),
          full(wo_g.shape), full(wo_m.shape), full(bo.shape),
      ],
      out_specs=pl.BlockSpec((blk,), lambda i: (i,)),
      out_shape=jax.ShapeDtypeStruct((B,), jnp.float32),
  )(uq, iq, gu, gi, mu, mi, W1a, W1b, b1, W2, b2, wo_g, wo_m, bo)


@jax.jit
def _neumf(user_idx, item_idx, gmf_user_emb, gmf_item_emb,
           mlp_user_emb, mlp_item_emb, W1, b1, W2, b2, Wo, bo):
  info = plsc.get_sparse_core_info()
  nw = info.num_cores * info.num_subcores
  sc = _make_sc_gather(info.num_cores, info.num_subcores, B // nw)

  uidx = user_idx.astype(jnp.int32)
  iidx = item_idx.astype(jnp.int32)
  uq = (uidx >> 13) & 3
  urow = ((uidx >> 15) << 13) + (uidx & 8191)
  iq = (iidx >> 13) & 3
  irow = ((iidx >> 15) << 13) + (iidx & 8191)

  gu_t = _relayout(gmf_user_emb)
  gi_t = _relayout(gmf_item_emb)
  mu_t = _relayout(mlp_user_emb)
  mi_t = _relayout(mlp_item_emb)

  gu, gi, mu, mi = sc(urow, irow, gu_t, gi_t, mu_t, mi_t)

  W1a, W1b = W1[:F], W1[F:]
  wo_g, wo_m = Wo[:F, 0], Wo[F:, 0]
  return _tc_mlp(uq.reshape(B, 1), iq.reshape(B, 1), gu, gi, mu, mi,
                 W1a, W1b, b1, W2, b2, wo_g, wo_m, bo)


def kernel(user_idx, item_idx, gmf_user_emb, gmf_item_emb,
           mlp_user_emb, mlp_item_emb, W1, b1, W2, b2, Wo, bo):
  return _neumf(user_idx, item_idx, gmf_user_emb, gmf_item_emb,
                mlp_user_emb, mlp_item_emb, W1, b1, W2, b2, Wo, bo)
